# baseline (speedup 1.0000x reference)
import functools
import math
import struct

import jax
import jax.numpy as jnp
from jax import lax
from jax.experimental import pallas as pl
from jax.experimental.pallas import tpu as pltpu
from jax.experimental.pallas import tpu_sc as plsc

_C = 81
_NP = 5000
_FD = 1024
_IMG = 1024.0
_ST = 0.05
_NT = 0.5
_DET = 100
_K = 200
_WXY = 10.0
_WWH = 5.0
_CLIP = float(math.log(1000.0 / 16.0))
_NEG = -1e10
_DEAD = -2e10
_NPAD = 5120
_NC1 = _C - 1


def _prep_body(logit_ref, probs_ref):
    x = logit_ref[...]
    m = jnp.max(x, axis=1, keepdims=True)
    e = jnp.exp(x - m)
    p = e / jnp.sum(e, axis=1, keepdims=True)
    probs_ref[...] = jnp.where(p > _ST, p, _NEG)


def _prep(class_logit):
    return pl.pallas_call(
        _prep_body,
        out_shape=jax.ShapeDtypeStruct((_NP, _C), jnp.float32),
    )(class_logit)


def _topk_body(s_ref, topv_ref, topi_ref, topg_ref, sc_ref):
    sc_ref[...] = s_ref[...]
    lane = lax.broadcasted_iota(jnp.int32, (_NC1, _NPAD), 1)
    l200 = lax.broadcasted_iota(jnp.int32, (_NC1, _K), 1)

    def body(i, carry):
        accv, acci = carry
        s = sc_ref[...]
        m = jnp.max(s, axis=1, keepdims=True)
        cand = jnp.where(s == m, lane, jnp.int32(2**30))
        idx = jnp.min(cand, axis=1, keepdims=True)
        sc_ref[...] = jnp.where(lane == idx, _DEAD, s)
        accv = jnp.where(l200 == i, m, accv)
        acci = jnp.where(l200 == i, idx, acci)
        return accv, acci

    accv, acci = lax.fori_loop(
        0, _K, body,
        (jnp.full((_NC1, _K), _DEAD, jnp.float32),
         jnp.zeros((_NC1, _K), jnp.int32)))
    topv_ref[...] = accv
    topi_ref[...] = acci
    cls = lax.broadcasted_iota(jnp.int32, (_NC1, _K), 0) + 1
    topg_ref[...] = acci * _C + cls


def _topk(scores):
    return pl.pallas_call(
        _topk_body,
        out_shape=[
            jax.ShapeDtypeStruct((_NC1, _K), jnp.float32),
            jax.ShapeDtypeStruct((_NC1, _K), jnp.int32),
            jax.ShapeDtypeStruct((_NC1, _K), jnp.int32),
        ],
        scratch_shapes=[pltpu.VMEM((_NC1, _NPAD), jnp.float32)],
    )(scores)


def _sc_gather(table, idx, b_per_w):
    n_idx, d = idx.shape[0], table.shape[1]
    mesh = plsc.VectorSubcoreMesh(core_axis_name="c", subcore_axis_name="s")

    @functools.partial(
        pl.kernel,
        mesh=mesh,
        out_type=jax.ShapeDtypeStruct((n_idx, d), jnp.float32),
        scratch_types=[
            pltpu.VMEM((b_per_w,), jnp.int32),
            pltpu.VMEM((b_per_w, d), jnp.float32),
            pltpu.SemaphoreType.DMA,
        ],
    )
    def k(table_hbm, idx_hbm, out_hbm, idx_v, rows_v, sem):
        wid = lax.axis_index("s") * 2 + lax.axis_index("c")
        base = wid * b_per_w
        pltpu.sync_copy(idx_hbm.at[pl.ds(base, b_per_w)], idx_v)
        pltpu.async_copy(table_hbm.at[idx_v], rows_v, sem).wait()
        pltpu.sync_copy(rows_v, out_hbm.at[pl.ds(base, b_per_w)])

    return k(table, idx)


def _boxdec_sc(rel_flat, pb_flat, topi_flat):
    mesh = plsc.VectorSubcoreMesh(core_axis_name="c", subcore_axis_name="s")

    @functools.partial(
        pl.kernel,
        mesh=mesh,
        compiler_params=pltpu.CompilerParams(needs_layout_passes=False),
        out_type=jax.ShapeDtypeStruct((_NC1 * _K * 4,), jnp.float32),
        scratch_types=[
            pltpu.VMEM((_NP * 4,), jnp.float32),
            pltpu.VMEM((_NP * 4,), jnp.float32),
            pltpu.VMEM((208,), jnp.int32),
            pltpu.VMEM((832,), jnp.float32),
        ],
    )
    def k(rel_hbm, pb_hbm, ti_hbm, out_hbm, pb_v, rel_v, ti_v, out_v):
        wid = lax.axis_index("s") * 2 + lax.axis_index("c")
        pltpu.sync_copy(pb_hbm, pb_v)
        lanes = lax.iota(jnp.int32, 16)
        for t in range(3):
            cls = wid + 32 * t

            @pl.when(cls < _NC1)
            def _():
                pltpu.sync_copy(rel_hbm.at[pl.ds(cls * (_NP * 4), _NP * 4)],
                                rel_v)
                pltpu.sync_copy(ti_hbm.at[pl.ds(cls * _K, _K)],
                                ti_v.at[pl.ds(0, _K)])
                for j in range(13):
                    n = ti_v[pl.ds(j * 16, 16)]
                    b = jnp.minimum(n, _NP - 1) * 4
                    px1 = plsc.load_gather(pb_v, [b])
                    py1 = plsc.load_gather(pb_v, [b + 1])
                    px2 = plsc.load_gather(pb_v, [b + 2])
                    py2 = plsc.load_gather(pb_v, [b + 3])
                    rdx = plsc.load_gather(rel_v, [b])
                    rdy = plsc.load_gather(rel_v, [b + 1])
                    rdw = plsc.load_gather(rel_v, [b + 2])
                    rdh = plsc.load_gather(rel_v, [b + 3])
                    widths = px2 - px1 + 1.0
                    heights = py2 - py1 + 1.0
                    ctr_x = px1 + 0.5 * widths
                    ctr_y = py1 + 0.5 * heights
                    dx = rdx / _WXY
                    dy = rdy / _WXY
                    dw = jnp.minimum(rdw / _WWH, _CLIP)
                    dh = jnp.minimum(rdh / _WWH, _CLIP)
                    pcx = dx * widths + ctr_x
                    pcy = dy * heights + ctr_y
                    pw = jnp.exp(dw) * widths
                    ph = jnp.exp(dh) * heights
                    hi = _IMG - 1.0
                    ox1 = jnp.minimum(jnp.maximum(pcx - 0.5 * pw, 0.0), hi)
                    oy1 = jnp.minimum(jnp.maximum(pcy - 0.5 * ph, 0.0), hi)
                    ox2 = jnp.minimum(
                        jnp.maximum(pcx + 0.5 * pw - 1.0, 0.0), hi)
                    oy2 = jnp.minimum(
                        jnp.maximum(pcy + 0.5 * ph - 1.0, 0.0), hi)
                    s4 = (j * 16 + lanes) * 4
                    plsc.store_scatter(out_v, [s4], ox1)
                    plsc.store_scatter(out_v, [s4 + 1], oy1)
                    plsc.store_scatter(out_v, [s4 + 2], ox2)
                    plsc.store_scatter(out_v, [s4 + 3], oy2)
                pltpu.sync_copy(out_v.at[pl.ds(0, _K * 4)],
                                out_hbm.at[pl.ds(cls * (_K * 4), _K * 4)])

    return k(rel_flat, pb_flat, topi_flat)


def _nms_body(bx_ref, vt_ref, it_ref, boxes_o, fv_o, lab_o, oidx_o,
              area_ref, keep_ref):
    x1 = bx_ref[0]
    y1 = bx_ref[1]
    x2 = bx_ref[2]
    y2 = bx_ref[3]
    v = vt_ref[...]
    area_ref[...] = (x2 - x1 + 1.0) * (y2 - y1 + 1.0)
    keep_ref[...] = (v > -1e9).astype(jnp.int32)
    area = area_ref[...]
    ridx = lax.broadcasted_iota(jnp.int32, (_K, _NC1), 0)

    def nms_step(i, _):
        x1i = bx_ref[0, pl.ds(i, 1), :]
        y1i = bx_ref[1, pl.ds(i, 1), :]
        x2i = bx_ref[2, pl.ds(i, 1), :]
        y2i = bx_ref[3, pl.ds(i, 1), :]
        ai = area_ref[pl.ds(i, 1), :]
        ki = keep_ref[pl.ds(i, 1), :] != 0
        ltx = jnp.maximum(x1i, x1)
        lty = jnp.maximum(y1i, y1)
        rbx = jnp.minimum(x2i, x2)
        rby = jnp.minimum(y2i, y2)
        w = jnp.maximum(rbx - ltx + 1.0, 0.0)
        h = jnp.maximum(rby - lty + 1.0, 0.0)
        inter = w * h
        iou = inter / (ai + area - inter)
        suppress = (iou > _NT) & ki
        keep = keep_ref[...] != 0
        keep_ref[...] = (keep & ~(suppress & (ridx > i))).astype(jnp.int32)
        return 0

    lax.fori_loop(0, _K, nms_step, 0)
    keep = keep_ref[...] != 0

    cidx = lax.broadcasted_iota(jnp.int32, (_K, _NC1), 1)
    flat = cidx * _K + ridx
    s0 = jnp.where(keep, v, _NEG)
    it = it_ref[...]
    itf = it.astype(jnp.float32)
    lane = lax.broadcasted_iota(jnp.int32, (1, 128), 1)

    def sel_step(k, carry):
        s, fva, xa1, ya1, xa2, ya2, laba, oia = carry
        m = jnp.max(s)
        cand = jnp.where(s == m, flat, jnp.int32(2**30))
        fsel = jnp.min(cand)
        onehot = flat == fsel
        s = jnp.where(onehot, _DEAD, s)
        sel = lane == k

        def pickf(arr):
            return jnp.sum(jnp.where(onehot, arr, 0.0))

        fva = jnp.where(sel, m, fva)
        xa1 = jnp.where(sel, pickf(x1), xa1)
        ya1 = jnp.where(sel, pickf(y1), ya1)
        xa2 = jnp.where(sel, pickf(x2), xa2)
        ya2 = jnp.where(sel, pickf(y2), ya2)
        laba = jnp.where(sel, fsel // _K + 1, laba)
        oia = jnp.where(sel, pickf(itf).astype(jnp.int32), oia)
        return s, fva, xa1, ya1, xa2, ya2, laba, oia

    z = jnp.zeros((1, 128), jnp.float32)
    zi = jnp.zeros((1, 128), jnp.int32)
    carry = lax.fori_loop(0, _DET, sel_step, (s0, z, z, z, z, z, zi, zi))
    _, fva, xa1, ya1, xa2, ya2, laba, oia = carry
    fv_o[...] = fva
    boxes_o[0:1] = xa1[None]
    boxes_o[1:2] = ya1[None]
    boxes_o[2:3] = xa2[None]
    boxes_o[3:4] = ya2[None]
    lab_o[...] = laba
    oidx_o[...] = oia


def _nms_final(bx, vt, it):
    return pl.pallas_call(
        _nms_body,
        out_shape=[
            jax.ShapeDtypeStruct((4, 1, 128), jnp.float32),
            jax.ShapeDtypeStruct((1, 128), jnp.float32),
            jax.ShapeDtypeStruct((1, 128), jnp.int32),
            jax.ShapeDtypeStruct((1, 128), jnp.int32),
        ],
        scratch_shapes=[
            pltpu.VMEM((_K, _NC1), jnp.float32),
            pltpu.VMEM((_K, _NC1), jnp.int32),
        ],
    )(bx, vt, it)


def kernel(class_logit, box_regression, proposal_boxes, features):
    probs = _prep(class_logit)

    scores = jnp.pad(probs[:, 1:].T, ((0, 0), (0, _NPAD - _NP)),
                     constant_values=_DEAD)
    topv, topi, _ = _topk(scores)

    rel_flat = jnp.transpose(box_regression.reshape(_NP, _C, 4),
                             (1, 0, 2))[1:].reshape(-1)
    rows = _boxdec_sc(rel_flat, proposal_boxes.reshape(-1),
                      topi.reshape(-1))

    bx = jnp.transpose(rows.reshape(_NC1, _K, 4), (2, 1, 0))
    boxes_o, fv_o, lab_o, oidx_o = _nms_final(bx, topv.T, topi.T)

    fpad = 256 - _DET
    fidx = jnp.concatenate(
        [oidx_o[0, :_DET],
         (jnp.arange(fpad, dtype=jnp.int32) * 37) % _NP])
    feats = _sc_gather(features, fidx, 256 // 32)[:_DET]

    final_boxes = boxes_o[:, 0, :_DET].T
    return final_boxes, fv_o[0, :_DET], lab_o[0, :_DET], feats


# speedup vs baseline: 1.6334x; 1.6334x over previous
import functools
import math

import jax
import jax.numpy as jnp
from jax import lax
from jax.experimental import pallas as pl
from jax.experimental.pallas import tpu as pltpu
from jax.experimental.pallas import tpu_sc as plsc

_C = 81
_NP = 5000
_FD = 1024
_IMG = 1024.0
_ST = 0.05
_NT = 0.5
_DET = 100
_K = 200
_WXY = 10.0
_WWH = 5.0
_CLIP = float(math.log(1000.0 / 16.0))
_NEG = -1e10
_DEAD = -2e10
_NPAD = 5120
_NC1 = _C - 1


def _prep_body(logit_ref, rel_ref, pb_ref, probs_ref, dec_ref):
    x = logit_ref[...]
    m = jnp.max(x, axis=1, keepdims=True)
    e = jnp.exp(x - m)
    p = e / jnp.sum(e, axis=1, keepdims=True)
    probs_ref[...] = jnp.where(p > _ST, p, _NEG)

    b = pb_ref[...]
    x1 = b[:, 0:1]
    y1 = b[:, 1:2]
    x2 = b[:, 2:3]
    y2 = b[:, 3:4]
    widths = x2 - x1 + 1.0
    heights = y2 - y1 + 1.0
    ctr_x = x1 + 0.5 * widths
    ctr_y = y1 + 0.5 * heights
    dx = rel_ref[0] / _WXY
    dy = rel_ref[1] / _WXY
    dw = jnp.minimum(rel_ref[2] / _WWH, _CLIP)
    dh = jnp.minimum(rel_ref[3] / _WWH, _CLIP)
    pred_ctr_x = dx * widths + ctr_x
    pred_ctr_y = dy * heights + ctr_y
    pred_w = jnp.exp(dw) * widths
    pred_h = jnp.exp(dh) * heights
    ox1 = pred_ctr_x - 0.5 * pred_w
    oy1 = pred_ctr_y - 0.5 * pred_h
    ox2 = pred_ctr_x + 0.5 * pred_w - 1.0
    oy2 = pred_ctr_y + 0.5 * pred_h - 1.0
    dec_ref[0] = jnp.clip(ox1, 0.0, _IMG - 1.0)
    dec_ref[1] = jnp.clip(oy1, 0.0, _IMG - 1.0)
    dec_ref[2] = jnp.clip(ox2, 0.0, _IMG - 1.0)
    dec_ref[3] = jnp.clip(oy2, 0.0, _IMG - 1.0)


def _prep(class_logit, rel4, proposal_boxes):
    rows = 1000
    grid = _NP // rows
    return pl.pallas_call(
        _prep_body,
        grid=(grid,),
        in_specs=[
            pl.BlockSpec((rows, _C), lambda i: (i, 0)),
            pl.BlockSpec((4, rows, _C), lambda i: (0, i, 0)),
            pl.BlockSpec((rows, 4), lambda i: (i, 0)),
        ],
        out_specs=[
            pl.BlockSpec((rows, _C), lambda i: (i, 0)),
            pl.BlockSpec((4, rows, _C), lambda i: (0, i, 0)),
        ],
        out_shape=[
            jax.ShapeDtypeStruct((_NP, _C), jnp.float32),
            jax.ShapeDtypeStruct((4, _NP, _C), jnp.float32),
        ],
    )(class_logit, rel4, proposal_boxes)


def _boxgather_body(dec_ref, ti_ref, out_ref):
    tcol = ti_ref[0]
    n_iota = lax.broadcasted_iota(jnp.int32, (_K, _NP), 1)
    oh = jnp.where(n_iota == tcol, 1.0, 0.0)
    out_ref[0] = jnp.dot(oh, dec_ref[0],
                         preferred_element_type=jnp.float32)


def _boxgather(dec2, topi3):
    return pl.pallas_call(
        _boxgather_body,
        grid=(_NC1,),
        in_specs=[
            pl.BlockSpec((1, _NP, 4), lambda c: (c + 1, 0, 0)),
            pl.BlockSpec((1, _K, 1), lambda c: (c, 0, 0)),
        ],
        out_specs=pl.BlockSpec((1, _K, 4), lambda c: (c, 0, 0)),
        out_shape=jax.ShapeDtypeStruct((_NC1, _K, 4), jnp.float32),
    )(dec2, topi3)


def _topk_body(s_ref, topv_ref, topi_ref, topg_ref, sc_ref):
    sc_ref[...] = s_ref[...]
    lane = lax.broadcasted_iota(jnp.int32, (_NC1, _NPAD), 1)
    l200 = lax.broadcasted_iota(jnp.int32, (_NC1, _K), 1)

    def body(i, carry):
        accv, acci = carry
        s = sc_ref[...]
        m = jnp.max(s, axis=1, keepdims=True)
        cand = jnp.where(s == m, lane, jnp.int32(2**30))
        idx = jnp.min(cand, axis=1, keepdims=True)
        sc_ref[...] = jnp.where(lane == idx, _DEAD, s)
        accv = jnp.where(l200 == i, m, accv)
        acci = jnp.where(l200 == i, idx, acci)
        return accv, acci

    accv, acci = lax.fori_loop(
        0, _K, body,
        (jnp.full((_NC1, _K), _DEAD, jnp.float32),
         jnp.zeros((_NC1, _K), jnp.int32)))
    topv_ref[...] = accv
    topi_ref[...] = acci
    cls = lax.broadcasted_iota(jnp.int32, (_NC1, _K), 0) + 1
    topg_ref[...] = acci * _C + cls


def _topk(scores):
    return pl.pallas_call(
        _topk_body,
        out_shape=[
            jax.ShapeDtypeStruct((_NC1, _K), jnp.float32),
            jax.ShapeDtypeStruct((_NC1, _K), jnp.int32),
            jax.ShapeDtypeStruct((_NC1, _K), jnp.int32),
        ],
        scratch_shapes=[pltpu.VMEM((_NC1, _NPAD), jnp.float32)],
    )(scores)


def _sc_gather(table, idx, b_per_w):
    n_idx, d = idx.shape[0], table.shape[1]
    mesh = plsc.VectorSubcoreMesh(core_axis_name="c", subcore_axis_name="s")

    @functools.partial(
        pl.kernel,
        mesh=mesh,
        out_type=jax.ShapeDtypeStruct((n_idx, d), jnp.float32),
        scratch_types=[
            pltpu.VMEM((b_per_w,), jnp.int32),
            pltpu.VMEM((b_per_w, d), jnp.float32),
            pltpu.SemaphoreType.DMA,
        ],
    )
    def k(table_hbm, idx_hbm, out_hbm, idx_v, rows_v, sem):
        wid = lax.axis_index("s") * 2 + lax.axis_index("c")
        base = wid * b_per_w
        pltpu.sync_copy(idx_hbm.at[pl.ds(base, b_per_w)], idx_v)
        pltpu.async_copy(table_hbm.at[idx_v], rows_v, sem).wait()
        pltpu.sync_copy(rows_v, out_hbm.at[pl.ds(base, b_per_w)])

    return k(table, idx)


def _nms_body(bx_ref, vt_ref, it_ref, boxes_o, fv_o, lab_o, oidx_o,
              area_ref, keep_ref):
    x1 = bx_ref[0]
    y1 = bx_ref[1]
    x2 = bx_ref[2]
    y2 = bx_ref[3]
    v = vt_ref[...]
    area_ref[...] = (x2 - x1 + 1.0) * (y2 - y1 + 1.0)
    keep_ref[...] = (v > -1e9).astype(jnp.int32)
    area = area_ref[...]
    ridx = lax.broadcasted_iota(jnp.int32, (_K, _NC1), 0)

    def nms_step(i, _):
        x1i = bx_ref[0, pl.ds(i, 1), :]
        y1i = bx_ref[1, pl.ds(i, 1), :]
        x2i = bx_ref[2, pl.ds(i, 1), :]
        y2i = bx_ref[3, pl.ds(i, 1), :]
        ai = area_ref[pl.ds(i, 1), :]
        ki = keep_ref[pl.ds(i, 1), :] != 0
        ltx = jnp.maximum(x1i, x1)
        lty = jnp.maximum(y1i, y1)
        rbx = jnp.minimum(x2i, x2)
        rby = jnp.minimum(y2i, y2)
        w = jnp.maximum(rbx - ltx + 1.0, 0.0)
        h = jnp.maximum(rby - lty + 1.0, 0.0)
        inter = w * h
        iou = inter / (ai + area - inter)
        suppress = (iou > _NT) & ki
        keep = keep_ref[...] != 0
        keep_ref[...] = (keep & ~(suppress & (ridx > i))).astype(jnp.int32)
        return 0

    lax.fori_loop(0, _K, nms_step, 0)
    keep = keep_ref[...] != 0

    cidx = lax.broadcasted_iota(jnp.int32, (_K, _NC1), 1)
    flat = cidx * _K + ridx
    s0 = jnp.where(keep, v, _NEG)
    it = it_ref[...]
    itf = it.astype(jnp.float32)
    lane = lax.broadcasted_iota(jnp.int32, (1, 128), 1)

    def sel_step(k, carry):
        s, fva, xa1, ya1, xa2, ya2, laba, oia = carry
        m = jnp.max(s)
        cand = jnp.where(s == m, flat, jnp.int32(2**30))
        fsel = jnp.min(cand)
        onehot = flat == fsel
        s = jnp.where(onehot, _DEAD, s)
        sel = lane == k

        def pickf(arr):
            return jnp.sum(jnp.where(onehot, arr, 0.0))

        fva = jnp.where(sel, m, fva)
        xa1 = jnp.where(sel, pickf(x1), xa1)
        ya1 = jnp.where(sel, pickf(y1), ya1)
        xa2 = jnp.where(sel, pickf(x2), xa2)
        ya2 = jnp.where(sel, pickf(y2), ya2)
        laba = jnp.where(sel, fsel // _K + 1, laba)
        oia = jnp.where(sel, pickf(itf).astype(jnp.int32), oia)
        return s, fva, xa1, ya1, xa2, ya2, laba, oia

    z = jnp.zeros((1, 128), jnp.float32)
    zi = jnp.zeros((1, 128), jnp.int32)
    carry = lax.fori_loop(0, _DET, sel_step, (s0, z, z, z, z, z, zi, zi))
    _, fva, xa1, ya1, xa2, ya2, laba, oia = carry
    fv_o[...] = fva
    boxes_o[0:1] = xa1[None]
    boxes_o[1:2] = ya1[None]
    boxes_o[2:3] = xa2[None]
    boxes_o[3:4] = ya2[None]
    lab_o[...] = laba
    oidx_o[...] = oia


def _nms_final(bx, vt, it):
    return pl.pallas_call(
        _nms_body,
        out_shape=[
            jax.ShapeDtypeStruct((4, 1, 128), jnp.float32),
            jax.ShapeDtypeStruct((1, 128), jnp.float32),
            jax.ShapeDtypeStruct((1, 128), jnp.int32),
            jax.ShapeDtypeStruct((1, 128), jnp.int32),
        ],
        scratch_shapes=[
            pltpu.VMEM((_K, _NC1), jnp.float32),
            pltpu.VMEM((_K, _NC1), jnp.int32),
        ],
    )(bx, vt, it)


def kernel(class_logit, box_regression, proposal_boxes, features):
    rel4 = jnp.transpose(box_regression.reshape(_NP, _C, 4), (2, 0, 1))
    probs, dec = _prep(class_logit, rel4, proposal_boxes)

    scores = jnp.pad(probs[:, 1:].T, ((0, 0), (0, _NPAD - _NP)),
                     constant_values=_DEAD)
    topv, topi, _ = _topk(scores)

    dec2 = jnp.transpose(dec, (2, 1, 0))
    rows = _boxgather(dec2, topi[:, :, None])
    rows = rows.reshape(-1)

    bx = jnp.transpose(rows.reshape(_NC1, _K, 4), (2, 1, 0))
    boxes_o, fv_o, lab_o, oidx_o = _nms_final(bx, topv.T, topi.T)

    fpad = 256 - _DET
    fidx = jnp.concatenate(
        [oidx_o[0, :_DET],
         (jnp.arange(fpad, dtype=jnp.int32) * 37) % _NP])
    feats = _sc_gather(features, fidx, 256 // 32)[:_DET]

    final_boxes = boxes_o[:, 0, :_DET].T
    return final_boxes, fv_o[0, :_DET], lab_o[0, :_DET], feats
